# baseline (speedup 1.0000x reference)
import functools

import jax
import jax.numpy as jnp
from jax import lax
from jax.experimental import pallas as pl
from jax.experimental.pallas import tpu as pltpu
from jax.experimental.pallas import tpu_sc as plsc

VOCAB = 100000
D = 128
B = 1024
K = 5

_NC = 2
_NS = 16
_NW = _NC * _NS
_B_PER_W = B // _NW

BV = 8192
NB = (VOCAB + BV - 1) // BV


def _sc_gather_body(emb_hbm, tok_hbm, out_hbm, idx_v, rows_v, sem):
    wid = lax.axis_index("s") * _NC + lax.axis_index("c")
    base = wid * _B_PER_W
    pltpu.sync_copy(tok_hbm.at[pl.ds(base, _B_PER_W)], idx_v)
    pltpu.async_copy(emb_hbm.at[idx_v], rows_v, sem).wait()
    pltpu.sync_copy(rows_v, out_hbm.at[pl.ds(base, _B_PER_W)])


def _gather_queries(emb_norm, token_id):
    mesh = plsc.VectorSubcoreMesh(core_axis_name="c", subcore_axis_name="s")
    k = functools.partial(
        pl.kernel,
        mesh=mesh,
        out_type=jax.ShapeDtypeStruct((B, D), jnp.float32),
        scratch_types=[
            pltpu.VMEM((_B_PER_W,), jnp.int32),
            pltpu.VMEM((_B_PER_W, D), jnp.float32),
            pltpu.SemaphoreType.DMA,
        ],
    )(_sc_gather_body)
    return k(emb_norm, token_id)


def _tc_body(q_raw_ref, emb_ref, vals_ref, idx_ref, qn_ref):
    i = pl.program_id(0)

    @pl.when(i == 0)
    def _():
        q = q_raw_ref[...]
        n = jnp.sqrt(jnp.sum(q * q, axis=1, keepdims=True))
        qn_ref[...] = q / jnp.maximum(n, 1e-12)

    s = lax.dot_general(
        qn_ref[...], emb_ref[...],
        dimension_numbers=(((1,), (1,)), ((), ())),
        preferred_element_type=jnp.float32,
    )
    revi = ((B - 1) - lax.broadcasted_iota(jnp.int32, s.shape, 0)
            ).astype(jnp.float32)
    neg_inf = jnp.float32(-jnp.inf)
    for j in range(K):
        m = jnp.max(s, axis=0)
        r = jnp.max(jnp.where(s == m[None, :], revi, -1.0), axis=0)
        vals_ref[j, :] = m
        idx_ref[j, :] = (jnp.float32(B - 1) - r).astype(jnp.int32)
        if j + 1 < K:
            s = jnp.where(revi == r[None, :], neg_inf, s)


def _topk_tc(q_raw, emb_norm):
    return pl.pallas_call(
        _tc_body,
        grid=(NB,),
        in_specs=[
            pl.BlockSpec((B, D), lambda i: (0, 0)),
            pl.BlockSpec((BV, D), lambda i: (i, 0)),
        ],
        out_specs=[
            pl.BlockSpec((K, BV), lambda i: (0, i)),
            pl.BlockSpec((K, BV), lambda i: (0, i)),
        ],
        out_shape=[
            jax.ShapeDtypeStruct((K, VOCAB), jnp.float32),
            jax.ShapeDtypeStruct((K, VOCAB), jnp.int32),
        ],
        scratch_shapes=[pltpu.VMEM((B, D), jnp.float32)],
    )(q_raw, emb_norm)


def kernel(emb_norm, token_id):
    q_raw = _gather_queries(emb_norm, token_id)
    top_vals, top_idx = _topk_tc(q_raw, emb_norm)
    return (top_idx, top_vals)


# speedup vs baseline: 1.2755x; 1.2755x over previous
import functools

import jax
import jax.numpy as jnp
from jax import lax
from jax.experimental import pallas as pl
from jax.experimental.pallas import tpu as pltpu
from jax.experimental.pallas import tpu_sc as plsc

VOCAB = 100000
D = 128
B = 1024
K = 5

_NC = 2
_NS = 16
_NW = _NC * _NS
_B_PER_W = B // _NW

BV = 3072
NB = (VOCAB + BV - 1) // BV


def _sc_gather_body(emb_hbm, tok_hbm, out_hbm, idx_v, rows_v, sem):
    wid = lax.axis_index("s") * _NC + lax.axis_index("c")
    base = wid * _B_PER_W
    pltpu.sync_copy(tok_hbm.at[pl.ds(base, _B_PER_W)], idx_v)
    pltpu.async_copy(emb_hbm.at[idx_v], rows_v, sem).wait()
    pltpu.sync_copy(rows_v, out_hbm.at[pl.ds(base, _B_PER_W)])


def _gather_queries(emb_norm, token_id):
    mesh = plsc.VectorSubcoreMesh(core_axis_name="c", subcore_axis_name="s")
    k = functools.partial(
        pl.kernel,
        mesh=mesh,
        out_type=jax.ShapeDtypeStruct((B, D), jnp.float32),
        scratch_types=[
            pltpu.VMEM((_B_PER_W,), jnp.int32),
            pltpu.VMEM((_B_PER_W, D), jnp.float32),
            pltpu.SemaphoreType.DMA,
        ],
    )(_sc_gather_body)
    return k(emb_norm, token_id)


def _tc_body(q_raw_ref, emb_ref, vals_ref, idx_ref, qn_ref):
    i = pl.program_id(0)

    @pl.when(i == 0)
    def _():
        q = q_raw_ref[...]
        n = jnp.sqrt(jnp.sum(q * q, axis=1, keepdims=True))
        qn_ref[...] = q / jnp.maximum(n, 1e-12)

    s = lax.dot_general(
        qn_ref[...], emb_ref[...],
        dimension_numbers=(((1,), (1,)), ((), ())),
        preferred_element_type=jnp.float32,
    )
    revi = ((B - 1) - lax.broadcasted_iota(jnp.int32, s.shape, 0)
            ).astype(jnp.float32)
    neg_inf = jnp.float32(-jnp.inf)
    for j in range(K):
        m = jnp.max(s, axis=0)
        r = jnp.max(jnp.where(s == m[None, :], revi, -1.0), axis=0)
        vals_ref[j, :] = m
        idx_ref[j, :] = (jnp.float32(B - 1) - r).astype(jnp.int32)
        if j + 1 < K:
            s = jnp.where(revi == r[None, :], neg_inf, s)


def _topk_tc(q_raw, emb_norm):
    return pl.pallas_call(
        _tc_body,
        grid=(NB,),
        in_specs=[
            pl.BlockSpec((B, D), lambda i: (0, 0)),
            pl.BlockSpec((BV, D), lambda i: (i, 0)),
        ],
        out_specs=[
            pl.BlockSpec((K, BV), lambda i: (0, i)),
            pl.BlockSpec((K, BV), lambda i: (0, i)),
        ],
        out_shape=[
            jax.ShapeDtypeStruct((K, VOCAB), jnp.float32),
            jax.ShapeDtypeStruct((K, VOCAB), jnp.int32),
        ],
        scratch_shapes=[pltpu.VMEM((B, D), jnp.float32)],
    )(q_raw, emb_norm)


def kernel(emb_norm, token_id):
    q_raw = _gather_queries(emb_norm, token_id)
    top_vals, top_idx = _topk_tc(q_raw, emb_norm)
    return (top_idx, top_vals)


# speedup vs baseline: 1.2856x; 1.0079x over previous
import functools

import jax
import jax.numpy as jnp
from jax import lax
from jax.experimental import pallas as pl
from jax.experimental.pallas import tpu as pltpu
from jax.experimental.pallas import tpu_sc as plsc

VOCAB = 100000
D = 128
B = 1024
K = 5

_NC = 2
_NS = 16
_NW = _NC * _NS
_B_PER_W = B // _NW

BV = 4096
NB = (VOCAB + BV - 1) // BV


def _sc_gather_body(emb_hbm, tok_hbm, out_hbm, idx_v, rows_v, sem):
    wid = lax.axis_index("s") * _NC + lax.axis_index("c")
    base = wid * _B_PER_W
    pltpu.sync_copy(tok_hbm.at[pl.ds(base, _B_PER_W)], idx_v)
    pltpu.async_copy(emb_hbm.at[idx_v], rows_v, sem).wait()
    pltpu.sync_copy(rows_v, out_hbm.at[pl.ds(base, _B_PER_W)])


def _gather_queries(emb_norm, token_id):
    mesh = plsc.VectorSubcoreMesh(core_axis_name="c", subcore_axis_name="s")
    k = functools.partial(
        pl.kernel,
        mesh=mesh,
        out_type=jax.ShapeDtypeStruct((B, D), jnp.float32),
        scratch_types=[
            pltpu.VMEM((_B_PER_W,), jnp.int32),
            pltpu.VMEM((_B_PER_W, D), jnp.float32),
            pltpu.SemaphoreType.DMA,
        ],
    )(_sc_gather_body)
    return k(emb_norm, token_id)


def _tc_body(q_raw_ref, emb_ref, vals_ref, idx_ref, qn_ref):
    i = pl.program_id(0)

    @pl.when(i == 0)
    def _():
        q = q_raw_ref[...]
        n = jnp.sqrt(jnp.sum(q * q, axis=1, keepdims=True))
        qn_ref[...] = q / jnp.maximum(n, 1e-12)

    s = lax.dot_general(
        qn_ref[...], emb_ref[...],
        dimension_numbers=(((1,), (1,)), ((), ())),
        preferred_element_type=jnp.float32,
    )
    revi = ((B - 1) - lax.broadcasted_iota(jnp.int32, s.shape, 0)
            ).astype(jnp.float32)
    neg_inf = jnp.float32(-jnp.inf)
    for j in range(K):
        m = jnp.max(s, axis=0)
        r = jnp.max(jnp.where(s == m[None, :], revi, -1.0), axis=0)
        vals_ref[j, :] = m
        idx_ref[j, :] = (jnp.float32(B - 1) - r).astype(jnp.int32)
        if j + 1 < K:
            s = jnp.where(revi == r[None, :], neg_inf, s)


def _topk_tc(q_raw, emb_norm):
    return pl.pallas_call(
        _tc_body,
        grid=(NB,),
        in_specs=[
            pl.BlockSpec((B, D), lambda i: (0, 0)),
            pl.BlockSpec((BV, D), lambda i: (i, 0)),
        ],
        out_specs=[
            pl.BlockSpec((K, BV), lambda i: (0, i)),
            pl.BlockSpec((K, BV), lambda i: (0, i)),
        ],
        out_shape=[
            jax.ShapeDtypeStruct((K, VOCAB), jnp.float32),
            jax.ShapeDtypeStruct((K, VOCAB), jnp.int32),
        ],
        scratch_shapes=[pltpu.VMEM((B, D), jnp.float32)],
    )(q_raw, emb_norm)


def kernel(emb_norm, token_id):
    q_raw = _gather_queries(emb_norm, token_id)
    top_vals, top_idx = _topk_tc(q_raw, emb_norm)
    return (top_idx, top_vals)
